# baseline (speedup 1.0000x reference)
import functools

import jax
import jax.numpy as jnp
from jax import lax
from jax.experimental import pallas as pl
from jax.experimental.pallas import tpu as pltpu
from jax.experimental.pallas import tpu_sc as plsc

VOCAB = 100000
D = 32
CTX = 20
B = 1024

NC = 2
NS = 16
NW = NC * NS
B_PER_W = B // NW

VB = 2048
NBLK = (VOCAB + VB - 1) // VB


def _sc_gather_sum(tokens, table):
    mesh = plsc.VectorSubcoreMesh(core_axis_name="c", subcore_axis_name="s")

    @functools.partial(
        pl.kernel,
        mesh=mesh,
        out_type=jax.ShapeDtypeStruct((B, D), jnp.float32),
        scratch_types=[
            pltpu.VMEM((CTX, B_PER_W), jnp.int32),
            pltpu.VMEM((CTX * B_PER_W, D), jnp.float32),
            pltpu.VMEM((B_PER_W, D), jnp.float32),
            pltpu.SemaphoreType.DMA,
        ],
        compiler_params=pltpu.CompilerParams(use_tc_tiling_on_sc=False),
    )
    def sc_kernel(idx_hbm, table_hbm, out_hbm, idx_v, rows_v, s_v, sem):
        wid = lax.axis_index("s") * NC + lax.axis_index("c")
        base = wid * B_PER_W
        pltpu.sync_copy(idx_hbm.at[:, pl.ds(base, B_PER_W)], idx_v)
        for c in range(CTX):
            pltpu.async_copy(
                table_hbm.at[idx_v.at[c]],
                rows_v.at[pl.ds(c * B_PER_W, B_PER_W)],
                sem,
            )
        for c in range(CTX):
            pltpu.make_async_copy(
                table_hbm.at[idx_v.at[c]],
                rows_v.at[pl.ds(c * B_PER_W, B_PER_W)],
                sem,
            ).wait()

        @pl.loop(0, B_PER_W)
        def _(g):
            for h in range(D // 16):
                sl = pl.ds(h * 16, 16)
                acc = rows_v[g, sl]
                for c in range(1, CTX):
                    acc = acc + rows_v[c * B_PER_W + g, sl]
                s_v[g, sl] = acc

        pltpu.sync_copy(s_v, out_hbm.at[pl.ds(base, B_PER_W)])

    return sc_kernel(tokens, table)


def _logits_t(wt_ref, s_ref, b_ref):
    wb = wt_ref[...].astype(jnp.bfloat16)
    sb = s_ref[...].astype(jnp.bfloat16)
    lt = lax.dot_general(
        wb, sb, (((0,), (1,)), ((), ())), preferred_element_type=jnp.float32
    )
    return lt + b_ref[...]


def _fused_body(wt_ref, s_ref, b_ref, out_ref, l_ref, n_ref):
    i = pl.program_id(0)
    lt = _logits_t(wt_ref, s_ref, b_ref)

    @pl.when(i == 0)
    def _():
        l_ref[...] = jnp.zeros_like(l_ref)

    @pl.when(i < NBLK - 1)
    def _():
        l_ref[...] += jnp.sum(jnp.exp(lt), axis=0, keepdims=True)

    @pl.when(i == NBLK - 1)
    def _():
        rows = lax.broadcasted_iota(jnp.int32, (VB, 1), 0) + (NBLK - 1) * VB
        em = jnp.where(rows < VOCAB, jnp.exp(lt), 0.0)
        n_ref[...] = jnp.log(l_ref[...] + jnp.sum(em, axis=0, keepdims=True))

    @pl.when(i >= NBLK)
    def _():
        out_ref[...] = lt - n_ref[...]


def kernel(tokens, embed_table, W, b):
    s = _sc_gather_sum(tokens.astype(jnp.int32), embed_table)
    wt = W.T
    bc = b.reshape(VOCAB, 1)

    def _blk(i):
        return jnp.where(i < NBLK, i, i - NBLK)

    out_t = pl.pallas_call(
        _fused_body,
        grid=(2 * NBLK,),
        in_specs=[
            pl.BlockSpec((D, VB), lambda i: (0, _blk(i))),
            pl.BlockSpec((B, D), lambda i: (0, 0)),
            pl.BlockSpec((VB, 1), lambda i: (_blk(i), 0)),
        ],
        out_specs=pl.BlockSpec(
            (VB, B), lambda i: (jnp.where(i < NBLK, 0, i - NBLK), 0)
        ),
        out_shape=jax.ShapeDtypeStruct((VOCAB, B), jnp.float32),
        scratch_shapes=[
            pltpu.VMEM((1, B), jnp.float32),
            pltpu.VMEM((1, B), jnp.float32),
        ],
        compiler_params=pltpu.CompilerParams(
            dimension_semantics=("arbitrary",)
        ),
    )(wt, s, bc)
    return out_t.T


# speedup vs baseline: 1.1114x; 1.1114x over previous
import functools

import jax
import jax.numpy as jnp
from jax import lax
from jax.experimental import pallas as pl
from jax.experimental.pallas import tpu as pltpu
from jax.experimental.pallas import tpu_sc as plsc

VOCAB = 100000
D = 32
CTX = 20
B = 1024

NC = 2
NS = 16
NW = NC * NS
B_PER_W = B // NW

VB = 2048
NBLK = (VOCAB + VB - 1) // VB


def _sc_gather_sum(tokens, table):
    mesh = plsc.VectorSubcoreMesh(core_axis_name="c", subcore_axis_name="s")

    @functools.partial(
        pl.kernel,
        mesh=mesh,
        out_type=jax.ShapeDtypeStruct((B, D), jnp.float32),
        scratch_types=[
            pltpu.VMEM((CTX, B_PER_W), jnp.int32),
            pltpu.VMEM((CTX * B_PER_W, D), jnp.float32),
            pltpu.VMEM((B_PER_W, D), jnp.float32),
            pltpu.SemaphoreType.DMA,
        ],
        compiler_params=pltpu.CompilerParams(use_tc_tiling_on_sc=False),
    )
    def sc_kernel(idx_hbm, table_hbm, out_hbm, idx_v, rows_v, s_v, sem):
        wid = lax.axis_index("s") * NC + lax.axis_index("c")
        base = wid * B_PER_W
        pltpu.sync_copy(idx_hbm.at[:, pl.ds(base, B_PER_W)], idx_v)
        for c in range(CTX):
            pltpu.async_copy(
                table_hbm.at[idx_v.at[c]],
                rows_v.at[pl.ds(c * B_PER_W, B_PER_W)],
                sem,
            )
        for c in range(CTX):
            pltpu.make_async_copy(
                table_hbm.at[idx_v.at[c]],
                rows_v.at[pl.ds(c * B_PER_W, B_PER_W)],
                sem,
            ).wait()

        @pl.loop(0, B_PER_W)
        def _(g):
            for h in range(D // 16):
                sl = pl.ds(h * 16, 16)
                acc = rows_v[g, sl]
                for c in range(1, CTX):
                    acc = acc + rows_v[c * B_PER_W + g, sl]
                s_v[g, sl] = acc

        pltpu.sync_copy(s_v, out_hbm.at[pl.ds(base, B_PER_W)])

    return sc_kernel(tokens, table)


def _logits_t(wt_ref, s_ref, b_ref):
    wb = wt_ref[...].astype(jnp.bfloat16)
    sb = s_ref[...].astype(jnp.bfloat16)
    lt = lax.dot_general(
        wb, sb, (((0,), (1,)), ((), ())), preferred_element_type=jnp.float32
    )
    return lt + b_ref[...]


def _pass1_body(wt_ref, s_ref, b_ref, norm_ref, l_ref):
    i = pl.program_id(0)

    @pl.when(i == 0)
    def _():
        l_ref[...] = jnp.zeros_like(l_ref)

    e = jnp.exp(_logits_t(wt_ref, s_ref, b_ref))

    @pl.when(i < NBLK - 1)
    def _():
        l_ref[...] += jnp.sum(e, axis=0, keepdims=True)

    @pl.when(i == NBLK - 1)
    def _():
        rows = lax.broadcasted_iota(jnp.int32, (VB, 1), 0) + i * VB
        em = jnp.where(rows < VOCAB, e, 0.0)
        norm_ref[...] = jnp.log(l_ref[...] + jnp.sum(em, axis=0, keepdims=True))


def _pass2_body(wt_ref, s_ref, b_ref, norm_ref, out_ref):
    out_ref[...] = _logits_t(wt_ref, s_ref, b_ref) - norm_ref[...]


def kernel(tokens, embed_table, W, b):
    s = _sc_gather_sum(tokens.astype(jnp.int32), embed_table)
    wt = W.T
    bc = b.reshape(VOCAB, 1)

    norm = pl.pallas_call(
        _pass1_body,
        grid=(NBLK,),
        in_specs=[
            pl.BlockSpec((D, VB), lambda i: (0, i)),
            pl.BlockSpec((B, D), lambda i: (0, 0)),
            pl.BlockSpec((VB, 1), lambda i: (i, 0)),
        ],
        out_specs=pl.BlockSpec((1, B), lambda i: (0, 0)),
        out_shape=jax.ShapeDtypeStruct((1, B), jnp.float32),
        scratch_shapes=[pltpu.VMEM((1, B), jnp.float32)],
        compiler_params=pltpu.CompilerParams(
            dimension_semantics=("arbitrary",)
        ),
    )(wt, s, bc)

    out_t = pl.pallas_call(
        _pass2_body,
        grid=(NBLK,),
        in_specs=[
            pl.BlockSpec((D, VB), lambda i: (0, i)),
            pl.BlockSpec((B, D), lambda i: (0, 0)),
            pl.BlockSpec((VB, 1), lambda i: (i, 0)),
            pl.BlockSpec((1, B), lambda i: (0, 0)),
        ],
        out_specs=pl.BlockSpec((VB, B), lambda i: (i, 0)),
        out_shape=jax.ShapeDtypeStruct((VOCAB, B), jnp.float32),
        compiler_params=pltpu.CompilerParams(
            dimension_semantics=("arbitrary",)
        ),
    )(wt, s, bc, norm)
    return out_t.T


# speedup vs baseline: 1.1244x; 1.0117x over previous
import functools

import jax
import jax.numpy as jnp
from jax import lax
from jax.experimental import pallas as pl
from jax.experimental.pallas import tpu as pltpu
from jax.experimental.pallas import tpu_sc as plsc

VOCAB = 100000
D = 32
CTX = 20
B = 1024

NC = 2
NS = 16
NW = NC * NS
B_PER_W = B // NW

VB = 2048
NBLK = (VOCAB + VB - 1) // VB


def _sc_gather_sum(tokens, table):
    mesh = plsc.VectorSubcoreMesh(core_axis_name="c", subcore_axis_name="s")

    @functools.partial(
        pl.kernel,
        mesh=mesh,
        out_type=jax.ShapeDtypeStruct((B, D), jnp.float32),
        scratch_types=[
            pltpu.VMEM((CTX, B_PER_W), jnp.int32),
            pltpu.VMEM((CTX * B_PER_W, D), jnp.float32),
            pltpu.VMEM((B_PER_W, D), jnp.float32),
            pltpu.SemaphoreType.DMA,
        ],
        compiler_params=pltpu.CompilerParams(use_tc_tiling_on_sc=False),
    )
    def sc_kernel(idx_hbm, table_hbm, out_hbm, idx_v, rows_v, s_v, sem):
        wid = lax.axis_index("s") * NC + lax.axis_index("c")
        base = wid * B_PER_W
        pltpu.sync_copy(idx_hbm.at[:, pl.ds(base, B_PER_W)], idx_v)
        for c in range(CTX):
            pltpu.async_copy(
                table_hbm.at[idx_v.at[c]],
                rows_v.at[pl.ds(c * B_PER_W, B_PER_W)],
                sem,
            )
        for c in range(CTX):
            pltpu.make_async_copy(
                table_hbm.at[idx_v.at[c]],
                rows_v.at[pl.ds(c * B_PER_W, B_PER_W)],
                sem,
            ).wait()

        @pl.loop(0, B_PER_W)
        def _(g):
            for h in range(D // 16):
                sl = pl.ds(h * 16, 16)
                acc = rows_v[g, sl]
                for c in range(1, CTX):
                    acc = acc + rows_v[c * B_PER_W + g, sl]
                s_v[g, sl] = acc

        pltpu.sync_copy(s_v, out_hbm.at[pl.ds(base, B_PER_W)])

    return sc_kernel(tokens, table)


def _logits_t(wt_ref, s_ref, b_ref):
    wb = wt_ref[...].astype(jnp.bfloat16)
    sb = s_ref[...].astype(jnp.bfloat16)
    lt = lax.dot_general(
        wb, sb, (((0,), (1,)), ((), ())), preferred_element_type=jnp.float32
    )
    return lt + b_ref[...]


def _pass1_body(wt_ref, s_ref, b_ref, norm_ref, l_ref):
    i = pl.program_id(0)

    @pl.when(i == 0)
    def _():
        l_ref[...] = jnp.zeros_like(l_ref)

    e = jnp.exp(_logits_t(wt_ref, s_ref, b_ref))

    @pl.when(i < NBLK - 1)
    def _():
        l_ref[...] += jnp.sum(e, axis=0, keepdims=True)

    @pl.when(i == NBLK - 1)
    def _():
        rows = lax.broadcasted_iota(jnp.int32, (VB, 1), 0) + i * VB
        em = jnp.where(rows < VOCAB, e, 0.0)
        norm_ref[...] = jnp.log(l_ref[...] + jnp.sum(em, axis=0, keepdims=True))


def _pass2_body(wt_ref, s_ref, b_ref, norm_ref, out_ref):
    out_ref[...] = _logits_t(wt_ref, s_ref, b_ref) - norm_ref[...]


def kernel(tokens, embed_table, W, b):
    s = _sc_gather_sum(tokens.astype(jnp.int32), embed_table)
    wt = W.T
    bc = b.reshape(VOCAB, 1)

    norm = pl.pallas_call(
        _pass1_body,
        grid=(NBLK,),
        in_specs=[
            pl.BlockSpec((D, VB), lambda i: (0, i)),
            pl.BlockSpec((B, D), lambda i: (0, 0)),
            pl.BlockSpec((VB, 1), lambda i: (i, 0)),
        ],
        out_specs=pl.BlockSpec((1, B), lambda i: (0, 0)),
        out_shape=jax.ShapeDtypeStruct((1, B), jnp.float32),
        scratch_shapes=[pltpu.VMEM((1, B), jnp.float32)],
        compiler_params=pltpu.CompilerParams(
            dimension_semantics=("arbitrary",)
        ),
    )(wt, s, bc)

    VB2 = 4096
    NBLK2 = (VOCAB + VB2 - 1) // VB2
    out_t = pl.pallas_call(
        _pass2_body,
        grid=(NBLK2,),
        in_specs=[
            pl.BlockSpec((D, VB2), lambda i: (0, i)),
            pl.BlockSpec((B, D), lambda i: (0, 0)),
            pl.BlockSpec((VB2, 1), lambda i: (i, 0)),
            pl.BlockSpec((1, B), lambda i: (0, 0)),
        ],
        out_specs=pl.BlockSpec((VB2, B), lambda i: (i, 0)),
        out_shape=jax.ShapeDtypeStruct((VOCAB, B), jnp.float32),
        compiler_params=pltpu.CompilerParams(
            dimension_semantics=("arbitrary",)
        ),
    )(wt, s, bc, norm)
    return out_t.T
